# baseline (speedup 1.0000x reference)
import functools

import jax
import jax.numpy as jnp
from jax import lax
from jax.experimental import pallas as pl
from jax.experimental.pallas import tpu as pltpu
from jax.experimental.pallas import tpu_sc as plsc

B = 16384
D_IN = 64
D_EMB = 2 * D_IN
NC = 2
NS = 16
NW = NC * NS
SUB = 128
NJ = B // (NW * SUB)
PW = NJ * SUB

BM = 1024
NB = B // BM
TCC = 128


def _sc_gather_body(table_hbm, lem_hbm, w_hbm, idx_v, rows_v,
                    sem_i, sem_g0, sem_g1, sem_w0, sem_w1):
    wid = lax.axis_index("s") * NC + lax.axis_index("c")
    base = wid * PW
    gsems = (sem_g0, sem_g1)
    wsems = (sem_w0, sem_w1)

    idx_copies = [
        pltpu.async_copy(lem_hbm.at[pl.ds(base + j * SUB, SUB)],
                         idx_v.at[j], sem_i)
        for j in range(NJ)
    ]
    for c in idx_copies:
        c.wait()

    def gather(j, b):
        return pltpu.async_copy(table_hbm.at[idx_v.at[j]], rows_v.at[b],
                                gsems[b])

    def writeback(j, b):
        return pltpu.async_copy(rows_v.at[b],
                                w_hbm.at[pl.ds(base + j * SUB, SUB)],
                                wsems[b])

    g = {0: gather(0, 0)}
    w = {}
    for j in range(NJ):
        b = j % 2
        g[j].wait()
        w[j] = writeback(j, b)
        if j + 1 < NJ:
            if j - 1 >= 0:
                w[j - 1].wait()
            g[j + 1] = gather(j + 1, (j + 1) % 2)
    w[NJ - 2].wait()
    w[NJ - 1].wait()


def _tc_dot_body(w_ref, e1t_ref, e2t_ref, o_ref):
    eye = (lax.broadcasted_iota(jnp.int32, (TCC, TCC), 0)
           == lax.broadcasted_iota(jnp.int32, (TCC, TCC), 1)).astype(jnp.float32)
    outs = []
    for c in range(BM // TCC):
        sl = pl.ds(c * TCC, TCC)
        wc = w_ref[sl, :]
        m = (jnp.dot(wc[:, :D_IN], e1t_ref[:, sl],
                     preferred_element_type=jnp.float32)
             + jnp.dot(wc[:, D_IN:], e2t_ref[:, sl],
                       preferred_element_type=jnp.float32))
        outs.append(jnp.sum(m * eye, axis=0))
    s = jnp.concatenate(outs)
    o_ref[...] = 1.0 / (1.0 + jnp.exp(-s))


@jax.jit
def _run(lemma_embs, lemmas, e1t, e2t):
    mesh = plsc.VectorSubcoreMesh(core_axis_name="c", subcore_axis_name="s")
    gathered = functools.partial(
        pl.kernel,
        mesh=mesh,
        compiler_params=pltpu.CompilerParams(needs_layout_passes=False),
        out_type=jax.ShapeDtypeStruct((B, D_EMB), jnp.float32),
        scratch_types=[
            pltpu.VMEM((NJ, SUB), jnp.int32),
            pltpu.VMEM((2, SUB, D_EMB), jnp.float32),
            pltpu.SemaphoreType.DMA,
            pltpu.SemaphoreType.DMA,
            pltpu.SemaphoreType.DMA,
            pltpu.SemaphoreType.DMA,
            pltpu.SemaphoreType.DMA,
        ],
    )(_sc_gather_body)(lemma_embs, lemmas)

    return pl.pallas_call(
        _tc_dot_body,
        grid=(NB,),
        in_specs=[
            pl.BlockSpec((BM, D_EMB), lambda i: (i, 0)),
            pl.BlockSpec((D_IN, BM), lambda i: (0, i)),
            pl.BlockSpec((D_IN, BM), lambda i: (0, i)),
        ],
        out_specs=pl.BlockSpec((BM,), lambda i: (i,)),
        out_shape=jax.ShapeDtypeStruct((B,), jnp.float32),
    )(gathered, e1t, e2t)


def kernel(emb1, emb2, lemmas, lemma_embs):
    return _run(lemma_embs, lemmas, emb1.T, emb2.T)


# speedup vs baseline: 1.1755x; 1.1755x over previous
import functools

import jax
import jax.numpy as jnp
from jax import lax
from jax.experimental import pallas as pl
from jax.experimental.pallas import tpu as pltpu
from jax.experimental.pallas import tpu_sc as plsc

B = 16384
D_IN = 64
D_EMB = 2 * D_IN
NC = 2
NS = 16
NW = NC * NS
SUB = 128
NJ = B // (NW * SUB)
PW = NJ * SUB

BM = 4096
NB = B // BM
TCC = 128


def _sc_gather_body(table_hbm, lem_hbm, w_hbm, idx_v, rows_v,
                    sem_i, sem_g0, sem_g1, sem_w0, sem_w1):
    wid = lax.axis_index("s") * NC + lax.axis_index("c")
    base = wid * PW
    gsems = (sem_g0, sem_g1)
    wsems = (sem_w0, sem_w1)

    idx_copies = [
        pltpu.async_copy(lem_hbm.at[pl.ds(base + j * SUB, SUB)],
                         idx_v.at[j], sem_i)
        for j in range(NJ)
    ]
    for c in idx_copies:
        c.wait()

    def gather(j, b):
        return pltpu.async_copy(table_hbm.at[idx_v.at[j]], rows_v.at[b],
                                gsems[b])

    def writeback(j, b):
        return pltpu.async_copy(rows_v.at[b],
                                w_hbm.at[pl.ds(base + j * SUB, SUB)],
                                wsems[b])

    g = {0: gather(0, 0)}
    w = {}
    for j in range(NJ):
        b = j % 2
        g[j].wait()
        w[j] = writeback(j, b)
        if j + 1 < NJ:
            if j - 1 >= 0:
                w[j - 1].wait()
            g[j + 1] = gather(j + 1, (j + 1) % 2)
    w[NJ - 2].wait()
    w[NJ - 1].wait()


def _tc_dot_body(w_ref, e1t_ref, e2t_ref, o_ref):
    eye = (lax.broadcasted_iota(jnp.int32, (TCC, TCC), 0)
           == lax.broadcasted_iota(jnp.int32, (TCC, TCC), 1)).astype(jnp.float32)
    outs = []
    for c in range(BM // TCC):
        sl = pl.ds(c * TCC, TCC)
        wc = w_ref[sl, :]
        m = (jnp.dot(wc[:, :D_IN], e1t_ref[:, sl],
                     preferred_element_type=jnp.float32)
             + jnp.dot(wc[:, D_IN:], e2t_ref[:, sl],
                       preferred_element_type=jnp.float32))
        outs.append(jnp.sum(m * eye, axis=0))
    s = jnp.concatenate(outs)
    o_ref[...] = 1.0 / (1.0 + jnp.exp(-s))


@jax.jit
def _run(lemma_embs, lemmas, e1t, e2t):
    mesh = plsc.VectorSubcoreMesh(core_axis_name="c", subcore_axis_name="s")
    gathered = functools.partial(
        pl.kernel,
        mesh=mesh,
        compiler_params=pltpu.CompilerParams(needs_layout_passes=False),
        out_type=jax.ShapeDtypeStruct((B, D_EMB), jnp.float32),
        scratch_types=[
            pltpu.VMEM((NJ, SUB), jnp.int32),
            pltpu.VMEM((2, SUB, D_EMB), jnp.float32),
            pltpu.SemaphoreType.DMA,
            pltpu.SemaphoreType.DMA,
            pltpu.SemaphoreType.DMA,
            pltpu.SemaphoreType.DMA,
            pltpu.SemaphoreType.DMA,
        ],
    )(_sc_gather_body)(lemma_embs, lemmas)

    return pl.pallas_call(
        _tc_dot_body,
        grid=(NB,),
        in_specs=[
            pl.BlockSpec((BM, D_EMB), lambda i: (i, 0)),
            pl.BlockSpec((D_IN, BM), lambda i: (0, i)),
            pl.BlockSpec((D_IN, BM), lambda i: (0, i)),
        ],
        out_specs=pl.BlockSpec((BM,), lambda i: (i,)),
        out_shape=jax.ShapeDtypeStruct((B,), jnp.float32),
    )(gathered, e1t, e2t)


def kernel(emb1, emb2, lemmas, lemma_embs):
    return _run(lemma_embs, lemmas, emb1.T, emb2.T)


# speedup vs baseline: 1.1809x; 1.0046x over previous
import functools

import jax
import jax.numpy as jnp
from jax import lax
from jax.experimental import pallas as pl
from jax.experimental.pallas import tpu as pltpu
from jax.experimental.pallas import tpu_sc as plsc

B = 16384
D_IN = 64
D_EMB = 2 * D_IN
NC = 2
NS = 16
NW = NC * NS
SUB = 128
NJ = B // (NW * SUB)
PW = NJ * SUB

BM = 8192
NB = B // BM
TCC = 128


def _sc_gather_body(table_hbm, lem_hbm, w_hbm, idx_v, rows_v,
                    sem_i, sem_g0, sem_g1, sem_w0, sem_w1):
    wid = lax.axis_index("s") * NC + lax.axis_index("c")
    base = wid * PW
    gsems = (sem_g0, sem_g1)
    wsems = (sem_w0, sem_w1)

    idx_copies = [
        pltpu.async_copy(lem_hbm.at[pl.ds(base + j * SUB, SUB)],
                         idx_v.at[j], sem_i)
        for j in range(NJ)
    ]
    for c in idx_copies:
        c.wait()

    def gather(j, b):
        return pltpu.async_copy(table_hbm.at[idx_v.at[j]], rows_v.at[b],
                                gsems[b])

    def writeback(j, b):
        return pltpu.async_copy(rows_v.at[b],
                                w_hbm.at[pl.ds(base + j * SUB, SUB)],
                                wsems[b])

    g = {0: gather(0, 0)}
    w = {}
    for j in range(NJ):
        b = j % 2
        g[j].wait()
        w[j] = writeback(j, b)
        if j + 1 < NJ:
            if j - 1 >= 0:
                w[j - 1].wait()
            g[j + 1] = gather(j + 1, (j + 1) % 2)
    w[NJ - 2].wait()
    w[NJ - 1].wait()


def _tc_dot_body(w_ref, e1t_ref, e2t_ref, o_ref):
    eye = (lax.broadcasted_iota(jnp.int32, (TCC, TCC), 0)
           == lax.broadcasted_iota(jnp.int32, (TCC, TCC), 1)).astype(jnp.float32)
    outs = []
    for c in range(BM // TCC):
        sl = pl.ds(c * TCC, TCC)
        wc = w_ref[sl, :]
        m = (jnp.dot(wc[:, :D_IN], e1t_ref[:, sl],
                     preferred_element_type=jnp.float32)
             + jnp.dot(wc[:, D_IN:], e2t_ref[:, sl],
                       preferred_element_type=jnp.float32))
        outs.append(jnp.sum(m * eye, axis=0))
    s = jnp.concatenate(outs)
    o_ref[...] = 1.0 / (1.0 + jnp.exp(-s))


@jax.jit
def _run(lemma_embs, lemmas, e1t, e2t):
    mesh = plsc.VectorSubcoreMesh(core_axis_name="c", subcore_axis_name="s")
    gathered = functools.partial(
        pl.kernel,
        mesh=mesh,
        compiler_params=pltpu.CompilerParams(needs_layout_passes=False),
        out_type=jax.ShapeDtypeStruct((B, D_EMB), jnp.float32),
        scratch_types=[
            pltpu.VMEM((NJ, SUB), jnp.int32),
            pltpu.VMEM((2, SUB, D_EMB), jnp.float32),
            pltpu.SemaphoreType.DMA,
            pltpu.SemaphoreType.DMA,
            pltpu.SemaphoreType.DMA,
            pltpu.SemaphoreType.DMA,
            pltpu.SemaphoreType.DMA,
        ],
    )(_sc_gather_body)(lemma_embs, lemmas)

    return pl.pallas_call(
        _tc_dot_body,
        grid=(NB,),
        in_specs=[
            pl.BlockSpec((BM, D_EMB), lambda i: (i, 0)),
            pl.BlockSpec((D_IN, BM), lambda i: (0, i)),
            pl.BlockSpec((D_IN, BM), lambda i: (0, i)),
        ],
        out_specs=pl.BlockSpec((BM,), lambda i: (i,)),
        out_shape=jax.ShapeDtypeStruct((B,), jnp.float32),
    )(gathered, e1t, e2t)


def kernel(emb1, emb2, lemmas, lemma_embs):
    return _run(lemma_embs, lemmas, emb1.T, emb2.T)
